# baseline (speedup 1.0000x reference)
import functools

import jax
import jax.numpy as jnp
from jax import lax
from jax.experimental import pallas as pl
from jax.experimental.pallas import tpu as pltpu
from jax.experimental.pallas import tpu_sc as plsc

N = 100000
N2 = 100096
D = 128
NCH = 16
CH = 8
NS = 16
NC = 2
NPT = N2 // NS
NDUMMY = 96
B = 128
R2, R1, R0 = 25, 13, 7


def _prep_edges(edges, r):
    e = edges.shape[1]
    epad = NS * r * B
    j = jnp.arange(epad - e, dtype=jnp.int32)
    src = jnp.concatenate([edges[0], j % NDUMMY])
    dst = jnp.concatenate([edges[1], N + (j % NDUMMY)])
    return src.reshape(NS, r, B), dst.reshape(NS, r, B)


def _body(xc, s2, d2, s1, d1, s0, d0, out,
          table, is2, id2, is1, id1, is0, id0, vals):
    c = lax.axis_index("c")
    s = lax.axis_index("s")

    pltpu.sync_copy(s2.at[s], is2)
    pltpu.sync_copy(d2.at[s], id2)
    pltpu.sync_copy(s1.at[s], is1)
    pltpu.sync_copy(d1.at[s], id1)
    pltpu.sync_copy(s0.at[s], is0)
    pltpu.sync_copy(d0.at[s], id0)

    for cc in range(NCH // NC):
        chunk = c * (NCH // NC) + cc
        pltpu.sync_copy(xc.at[chunk, pl.ds(s * NPT, NPT)],
                        table.at[pl.ds(s * NPT, NPT)])
        plsc.subcore_barrier()

        for isx, idx, r_rows in ((is2, id2, R2), (is1, id1, R1),
                                 (is0, id0, R0)):
            def gather_row(r, _, isx=isx):
                pltpu.sync_copy(table.at[isx.at[r]],
                                vals.at[pl.ds(r * B, B)])
                return 0

            lax.fori_loop(0, r_rows, gather_row, 0)
            plsc.subcore_barrier()

            def scatter_row(r, _, idx=idx):
                pltpu.sync_copy(vals.at[pl.ds(r * B, B)],
                                table.at[idx.at[r]], add=True)
                return 0

            lax.fori_loop(0, r_rows, scatter_row, 0)
            plsc.subcore_barrier()

        pltpu.sync_copy(table.at[pl.ds(s * NPT, NPT)],
                        out.at[chunk, pl.ds(s * NPT, NPT)])


@jax.jit
def kernel(x, edges0, edges1, edges2):
    xc = x.reshape(NCH, CH, N).transpose(0, 2, 1)
    xc = jnp.pad(xc, ((0, 0), (0, N2 - N), (0, 0)))
    s2, d2 = _prep_edges(edges2, R2)
    s1, d1 = _prep_edges(edges1, R1)
    s0, d0 = _prep_edges(edges0, R0)

    run = pl.kernel(
        _body,
        out_type=jax.ShapeDtypeStruct((NCH, N2, CH), jnp.float32),
        mesh=plsc.VectorSubcoreMesh(core_axis_name="c",
                                    subcore_axis_name="s"),
        compiler_params=pltpu.CompilerParams(use_tc_tiling_on_sc=False),
        scratch_types=[
            pltpu.VMEM_SHARED((N2, CH), jnp.float32),
            pltpu.VMEM((R2, B), jnp.int32),
            pltpu.VMEM((R2, B), jnp.int32),
            pltpu.VMEM((R1, B), jnp.int32),
            pltpu.VMEM((R1, B), jnp.int32),
            pltpu.VMEM((R0, B), jnp.int32),
            pltpu.VMEM((R0, B), jnp.int32),
            pltpu.VMEM((R2 * B, CH), jnp.float32),
        ],
    )
    out = run(xc, s2, d2, s1, d1, s0, d0)
    return out[:, :N, :].transpose(0, 2, 1).reshape(D, N)


# speedup vs baseline: 1.0210x; 1.0210x over previous
import functools

import jax
import jax.numpy as jnp
from jax import lax
from jax.experimental import pallas as pl
from jax.experimental.pallas import tpu as pltpu
from jax.experimental.pallas import tpu_sc as plsc

N = 100000
N2 = 100096
D = 128
NCH = 16
CH = 8
NS = 16
NC = 2
NPT = N2 // NS
NDUMMY = 96
B = 128
R2, R1, R0 = 25, 13, 7


def _prep_edges(edges, r):
    e = edges.shape[1]
    epad = NS * r * B
    j = jnp.arange(epad - e, dtype=jnp.int32)
    src = jnp.concatenate([edges[0], j % NDUMMY])
    dst = jnp.concatenate([edges[1], N + (j % NDUMMY)])
    return src.reshape(NS, r, B), dst.reshape(NS, r, B)


def _body(xc, s2, d2, s1, d1, s0, d0, out,
          table, is2, id2, is1, id1, is0, id0, vals, sem):
    c = lax.axis_index("c")
    s = lax.axis_index("s")

    pltpu.sync_copy(s2.at[s], is2)
    pltpu.sync_copy(d2.at[s], id2)
    pltpu.sync_copy(s1.at[s], is1)
    pltpu.sync_copy(d1.at[s], id1)
    pltpu.sync_copy(s0.at[s], is0)
    pltpu.sync_copy(d0.at[s], id0)

    for cc in range(NCH // NC):
        chunk = c * (NCH // NC) + cc
        pltpu.sync_copy(xc.at[chunk, pl.ds(s * NPT, NPT)],
                        table.at[pl.ds(s * NPT, NPT)])
        plsc.subcore_barrier()

        for isx, idx, r_rows in ((is2, id2, R2), (is1, id1, R1),
                                 (is0, id0, R0)):
            def gather_row(r, _, isx=isx):
                pltpu.async_copy(table.at[isx.at[r]],
                                 vals.at[pl.ds(r * B, B)], sem)
                return 0

            lax.fori_loop(0, r_rows, gather_row, 0)
            pltpu.make_async_copy(xc.at[0, pl.ds(0, r_rows * B)],
                                  vals.at[pl.ds(0, r_rows * B)],
                                  sem).wait()
            plsc.subcore_barrier()

            def scatter_row(r, _, idx=idx):
                pltpu.async_copy(vals.at[pl.ds(r * B, B)],
                                 table.at[idx.at[r]], sem, add=True)
                return 0

            lax.fori_loop(0, r_rows, scatter_row, 0)
            pltpu.make_async_copy(xc.at[0, pl.ds(0, r_rows * B)],
                                  vals.at[pl.ds(0, r_rows * B)],
                                  sem).wait()
            plsc.subcore_barrier()

        pltpu.sync_copy(table.at[pl.ds(s * NPT, NPT)],
                        out.at[chunk, pl.ds(s * NPT, NPT)])


@jax.jit
def kernel(x, edges0, edges1, edges2):
    xc = x.reshape(NCH, CH, N).transpose(0, 2, 1)
    xc = jnp.pad(xc, ((0, 0), (0, N2 - N), (0, 0)))
    s2, d2 = _prep_edges(edges2, R2)
    s1, d1 = _prep_edges(edges1, R1)
    s0, d0 = _prep_edges(edges0, R0)

    run = pl.kernel(
        _body,
        out_type=jax.ShapeDtypeStruct((NCH, N2, CH), jnp.float32),
        mesh=plsc.VectorSubcoreMesh(core_axis_name="c",
                                    subcore_axis_name="s"),
        compiler_params=pltpu.CompilerParams(use_tc_tiling_on_sc=False),
        scratch_types=[
            pltpu.VMEM_SHARED((N2, CH), jnp.float32),
            pltpu.VMEM((R2, B), jnp.int32),
            pltpu.VMEM((R2, B), jnp.int32),
            pltpu.VMEM((R1, B), jnp.int32),
            pltpu.VMEM((R1, B), jnp.int32),
            pltpu.VMEM((R0, B), jnp.int32),
            pltpu.VMEM((R0, B), jnp.int32),
            pltpu.VMEM((R2 * B, CH), jnp.float32),
            pltpu.SemaphoreType.DMA,
        ],
    )
    out = run(xc, s2, d2, s1, d1, s0, d0)
    return out[:, :N, :].transpose(0, 2, 1).reshape(D, N)


# speedup vs baseline: 2.6410x; 2.5867x over previous
import jax
import jax.numpy as jnp
from jax import lax
from jax.experimental import pallas as pl
from jax.experimental.pallas import tpu as pltpu
from jax.experimental.pallas import tpu_sc as plsc

N = 100000
N2 = 100096
D = 128
NCH = 16
CH = 8
NS = 16
NC = 2
NPT = N2 // NS
NDUMMY = 96
B = 128
TW = 256
R2, R1, R0 = 25, 13, 7


def _prep_edges(edges, r):
    e = edges.shape[1]
    epad = NS * r * B
    j = jnp.arange(epad - e, dtype=jnp.int32)
    src = jnp.concatenate([edges[0], j % NDUMMY])
    dst = jnp.concatenate([edges[1], N + (j % NDUMMY)])
    return src.reshape(NS * r, B), dst.reshape(NS * r, B)


def _tin_body(x_ref, o_ref):
    o_ref[...] = x_ref[...].T


def _tout_body(y_ref, o_ref):
    o_ref[...] = y_ref[...].T


def _transpose_in(x):
    return pl.pallas_call(
        _tin_body,
        grid=(N2 // TW,),
        in_specs=[pl.BlockSpec((D, TW), lambda i: (0, i))],
        out_specs=pl.BlockSpec((TW, D), lambda i: (i, 0)),
        out_shape=jax.ShapeDtypeStruct((N2, D), jnp.float32),
    )(x)


def _transpose_out(yt):
    return pl.pallas_call(
        _tout_body,
        grid=(N2 // TW,),
        in_specs=[pl.BlockSpec((TW, D), lambda i: (i, 0))],
        out_specs=pl.BlockSpec((D, TW), lambda i: (0, i)),
        out_shape=jax.ShapeDtypeStruct((D, N), jnp.float32),
    )(yt)


def _body(xt, s2, d2, s1, d1, s0, d0, out,
          table, is2, id2, is1, id1, is0, id0, vals, sem):
    c = lax.axis_index("c")
    s = lax.axis_index("s")

    pltpu.sync_copy(s2.at[pl.ds(s * R2, R2)], is2)
    pltpu.sync_copy(d2.at[pl.ds(s * R2, R2)], id2)
    pltpu.sync_copy(s1.at[pl.ds(s * R1, R1)], is1)
    pltpu.sync_copy(d1.at[pl.ds(s * R1, R1)], id1)
    pltpu.sync_copy(s0.at[pl.ds(s * R0, R0)], is0)
    pltpu.sync_copy(d0.at[pl.ds(s * R0, R0)], id0)

    for cc in range(NCH // NC):
        chunk = c * (NCH // NC) + cc
        pltpu.sync_copy(xt.at[pl.ds(s * NPT, NPT), pl.ds(chunk * CH, CH)],
                        table.at[pl.ds(s * NPT, NPT)])
        plsc.subcore_barrier()

        for isx, idx, r_rows in ((is2, id2, R2), (is1, id1, R1),
                                 (is0, id0, R0)):
            def gather_row(r, _, isx=isx):
                pltpu.async_copy(table.at[isx.at[r]],
                                 vals.at[pl.ds(r * B, B)], sem)
                return 0

            lax.fori_loop(0, r_rows, gather_row, 0)
            pltpu.make_async_copy(xt.at[pl.ds(0, r_rows * B), pl.ds(0, CH)],
                                  vals.at[pl.ds(0, r_rows * B)],
                                  sem).wait()
            plsc.subcore_barrier()

            def scatter_row(r, _, idx=idx):
                pltpu.async_copy(vals.at[pl.ds(r * B, B)],
                                 table.at[idx.at[r]], sem, add=True)
                return 0

            lax.fori_loop(0, r_rows, scatter_row, 0)
            pltpu.make_async_copy(xt.at[pl.ds(0, r_rows * B), pl.ds(0, CH)],
                                  vals.at[pl.ds(0, r_rows * B)],
                                  sem).wait()
            plsc.subcore_barrier()

        pltpu.sync_copy(table.at[pl.ds(s * NPT, NPT)],
                        out.at[pl.ds(s * NPT, NPT), pl.ds(chunk * CH, CH)])


@jax.jit
def kernel(x, edges0, edges1, edges2):
    xt = _transpose_in(x)
    s2, d2 = _prep_edges(edges2, R2)
    s1, d1 = _prep_edges(edges1, R1)
    s0, d0 = _prep_edges(edges0, R0)

    run = pl.kernel(
        _body,
        out_type=jax.ShapeDtypeStruct((N2, D), jnp.float32),
        mesh=plsc.VectorSubcoreMesh(core_axis_name="c",
                                    subcore_axis_name="s"),
        compiler_params=pltpu.CompilerParams(use_tc_tiling_on_sc=False),
        scratch_types=[
            pltpu.VMEM_SHARED((N2, CH), jnp.float32),
            pltpu.VMEM((R2, B), jnp.int32),
            pltpu.VMEM((R2, B), jnp.int32),
            pltpu.VMEM((R1, B), jnp.int32),
            pltpu.VMEM((R1, B), jnp.int32),
            pltpu.VMEM((R0, B), jnp.int32),
            pltpu.VMEM((R0, B), jnp.int32),
            pltpu.VMEM((R2 * B, CH), jnp.float32),
            pltpu.SemaphoreType.DMA,
        ],
    )
    yt = run(xt, s2, d2, s1, d1, s0, d0)
    return _transpose_out(yt)


# speedup vs baseline: 5.9125x; 2.2387x over previous
import jax
import jax.numpy as jnp
from jax import lax
from jax.experimental import pallas as pl
from jax.experimental.pallas import tpu as pltpu
from jax.experimental.pallas import tpu_sc as plsc

N = 100000
N2 = 100352
D = 128
NCH = 16
CH = 8
NS = 16
NC = 2
NPT = N2 // NS
NDUMMY = 352
B = 128
SR = NPT // B
SH1, SH2 = 25, 24
TW = 2048
R2, R1, R0 = 25, 13, 7


def _prep_edges(edges, r):
    e = edges.shape[1]
    epad = NS * r * B
    j = jnp.arange(epad - e, dtype=jnp.int32)
    src = jnp.concatenate([edges[0], j % 128])
    dst = jnp.concatenate([edges[1], N + (j % NDUMMY)])
    return src.reshape(NS * r, B), dst.reshape(NS * r, B)


def _tin_body(x_ref, o_ref):
    o_ref[...] = x_ref[...].T


def _tout_body(y_ref, o_ref):
    o_ref[...] = y_ref[...].T


def _transpose_in(x):
    return pl.pallas_call(
        _tin_body,
        grid=(N2 // TW,),
        in_specs=[pl.BlockSpec((D, TW), lambda i: (0, i))],
        out_specs=pl.BlockSpec((TW, D), lambda i: (i, 0)),
        out_shape=jax.ShapeDtypeStruct((N2, D), jnp.float32),
    )(x)


def _transpose_out(yt):
    return pl.pallas_call(
        _tout_body,
        grid=(N2 // TW,),
        in_specs=[pl.BlockSpec((TW, D), lambda i: (i, 0))],
        out_specs=pl.BlockSpec((D, TW), lambda i: (0, i)),
        out_shape=jax.ShapeDtypeStruct((D, N), jnp.float32),
    )(yt)


def _body(xtv, sidx_all, s2, d2, s1, d1, s0, d0, out,
          table, sidx, is2, id2, is1, id1, is0, id0, vals, sem):
    c = lax.axis_index("c")
    s = lax.axis_index("s")

    pltpu.sync_copy(s2.at[pl.ds(s * R2, R2)], is2)
    pltpu.sync_copy(d2.at[pl.ds(s * R2, R2)], id2)
    pltpu.sync_copy(s1.at[pl.ds(s * R1, R1)], is1)
    pltpu.sync_copy(d1.at[pl.ds(s * R1, R1)], id1)
    pltpu.sync_copy(s0.at[pl.ds(s * R0, R0)], is0)
    pltpu.sync_copy(d0.at[pl.ds(s * R0, R0)], id0)

    for cc in range(NCH // NC):
        chunk = c * (NCH // NC) + cc
        pltpu.sync_copy(sidx_all.at[pl.ds(chunk * NS * SR + s * SR, SR)],
                        sidx)

        for ph, (p0, pn) in enumerate(((0, SH1), (SH1, SH2))):
            def stage_row(r, _, p0=p0):
                pltpu.async_copy(xtv.at[sidx.at[p0 + r]],
                                 vals.at[pl.ds(r * B, B)], sem)
                return 0

            lax.fori_loop(0, pn, stage_row, 0)
            pltpu.make_async_copy(xtv.at[pl.ds(0, pn * B)],
                                  vals.at[pl.ds(0, pn * B)], sem).wait()
            pltpu.sync_copy(vals.at[pl.ds(0, pn * B)],
                            table.at[pl.ds(s * NPT + p0 * B, pn * B)])
        plsc.subcore_barrier()

        for isx, idx, r_rows in ((is2, id2, R2), (is1, id1, R1),
                                 (is0, id0, R0)):
            def gather_row(r, _, isx=isx):
                pltpu.async_copy(table.at[isx.at[r]],
                                 vals.at[pl.ds(r * B, B)], sem)
                return 0

            lax.fori_loop(0, r_rows, gather_row, 0)
            pltpu.make_async_copy(xtv.at[pl.ds(0, r_rows * B)],
                                  vals.at[pl.ds(0, r_rows * B)],
                                  sem).wait()
            plsc.subcore_barrier()

            def scatter_row(r, _, idx=idx):
                pltpu.async_copy(vals.at[pl.ds(r * B, B)],
                                 table.at[idx.at[r]], sem, add=True)
                return 0

            lax.fori_loop(0, r_rows, scatter_row, 0)
            pltpu.make_async_copy(xtv.at[pl.ds(0, r_rows * B)],
                                  vals.at[pl.ds(0, r_rows * B)],
                                  sem).wait()
            plsc.subcore_barrier()

        for ph, (p0, pn) in enumerate(((0, SH1), (SH1, SH2))):
            pltpu.sync_copy(table.at[pl.ds(s * NPT + p0 * B, pn * B)],
                            vals.at[pl.ds(0, pn * B)])

            def out_row(r, _, p0=p0):
                pltpu.async_copy(vals.at[pl.ds(r * B, B)],
                                 out.at[sidx.at[p0 + r]], sem)
                return 0

            lax.fori_loop(0, pn, out_row, 0)
            pltpu.make_async_copy(xtv.at[pl.ds(0, pn * B)],
                                  vals.at[pl.ds(0, pn * B)], sem).wait()


@jax.jit
def kernel(x, edges0, edges1, edges2):
    xt = _transpose_in(x)
    xtv = xt.reshape(N2 * NS, CH)
    iot = (jnp.arange(N2, dtype=jnp.int32) * NS).reshape(NS * SR, B)
    sidx_all = (iot[None] + jnp.arange(NCH, dtype=jnp.int32)[:, None, None]
                ).reshape(NCH * NS * SR, B)
    s2, d2 = _prep_edges(edges2, R2)
    s1, d1 = _prep_edges(edges1, R1)
    s0, d0 = _prep_edges(edges0, R0)

    run = pl.kernel(
        _body,
        out_type=jax.ShapeDtypeStruct((N2 * NS, CH), jnp.float32),
        mesh=plsc.VectorSubcoreMesh(core_axis_name="c",
                                    subcore_axis_name="s"),
        compiler_params=pltpu.CompilerParams(use_tc_tiling_on_sc=False),
        scratch_types=[
            pltpu.VMEM_SHARED((N2, CH), jnp.float32),
            pltpu.VMEM((SR, B), jnp.int32),
            pltpu.VMEM((R2, B), jnp.int32),
            pltpu.VMEM((R2, B), jnp.int32),
            pltpu.VMEM((R1, B), jnp.int32),
            pltpu.VMEM((R1, B), jnp.int32),
            pltpu.VMEM((R0, B), jnp.int32),
            pltpu.VMEM((R0, B), jnp.int32),
            pltpu.VMEM((R2 * B, CH), jnp.float32),
            pltpu.SemaphoreType.DMA,
        ],
    )
    yt = run(xtv, sidx_all, s2, d2, s1, d1, s0, d0)
    return _transpose_out(yt.reshape(N2, D))
